# baseline (speedup 1.0000x reference)
import jax
import jax.numpy as jnp
from jax.experimental import pallas as pl
from jax.experimental.pallas import tpu as pltpu

BATCH = 1024
STACK = 100000
FEAT = 128
KEEP = STACK - BATCH


N_CHUNKS = 32
CHUNK = KEEP // N_CHUNKS


def _shift_kernel(x_ref, stack_ref, out_ref, sems, sem_new):
    copies = []
    for i in range(N_CHUNKS):
        c = pltpu.make_async_copy(
            stack_ref.at[pl.ds(BATCH + i * CHUNK, CHUNK), :],
            out_ref.at[pl.ds(i * CHUNK, CHUNK), :],
            sems.at[i],
        )
        c.start()
        copies.append(c)
    new = pltpu.make_async_copy(
        x_ref,
        out_ref.at[pl.ds(KEEP, BATCH), :],
        sem_new,
    )
    new.start()
    for c in copies:
        c.wait()
    new.wait()


def kernel(x, latent_stack):
    return pl.pallas_call(
        _shift_kernel,
        out_shape=jax.ShapeDtypeStruct((STACK, FEAT), jnp.float32),
        in_specs=[
            pl.BlockSpec(memory_space=pl.ANY),
            pl.BlockSpec(memory_space=pl.ANY),
        ],
        out_specs=pl.BlockSpec(memory_space=pl.ANY),
        scratch_shapes=[
            pltpu.SemaphoreType.DMA((N_CHUNKS,)),
            pltpu.SemaphoreType.DMA,
        ],
    )(x, latent_stack)


# speedup vs baseline: 21.6848x; 21.6848x over previous
import jax
import jax.numpy as jnp
from jax.experimental import pallas as pl
from jax.experimental.pallas import tpu as pltpu

BATCH = 1024
STACK = 100000
FEAT = 128
KEEP = STACK - BATCH

BLK = BATCH
NBLK = pl.cdiv(STACK, BLK)
SPLIT = STACK - 97 * BLK


def _shift_kernel(x_ref, st_ref, out_ref):
    i = pl.program_id(0)

    @pl.when(i <= NBLK - 3)
    def _pure_shift():
        out_ref[...] = st_ref[...]

    @pl.when(i == NBLK - 2)
    def _mixed():
        out_ref[0:SPLIT, :] = st_ref[0:SPLIT, :]
        out_ref[SPLIT:BLK, :] = x_ref[0 : BLK - SPLIT, :]

    @pl.when(i == NBLK - 1)
    def _tail():
        out_ref[0:SPLIT, :] = x_ref[BLK - SPLIT : BLK, :]


def kernel(x, latent_stack):
    return pl.pallas_call(
        _shift_kernel,
        grid=(NBLK,),
        out_shape=jax.ShapeDtypeStruct((STACK, FEAT), jnp.float32),
        in_specs=[
            pl.BlockSpec((BATCH, FEAT), lambda i: (0, 0)),
            pl.BlockSpec((BLK, FEAT), lambda i: (jnp.minimum(i + 1, NBLK - 1), 0)),
        ],
        out_specs=pl.BlockSpec((BLK, FEAT), lambda i: (i, 0)),
    )(x, latent_stack)


# speedup vs baseline: 21.6991x; 1.0007x over previous
import jax
import jax.numpy as jnp
from jax.experimental import pallas as pl
from jax.experimental.pallas import tpu as pltpu

BATCH = 1024
STACK = 100000
FEAT = 128
KEEP = STACK - BATCH

BLK = BATCH
NBLK = pl.cdiv(STACK, BLK)
SPLIT = STACK - 97 * BLK


def _shift_kernel(x_ref, st_ref, out_ref):
    i = pl.program_id(0)

    @pl.when(i <= NBLK - 3)
    def _pure_shift():
        out_ref[...] = st_ref[...]

    @pl.when(i == NBLK - 2)
    def _mixed():
        out_ref[0:SPLIT, :] = st_ref[0:SPLIT, :]
        out_ref[SPLIT:BLK, :] = x_ref[0 : BLK - SPLIT, :]

    @pl.when(i == NBLK - 1)
    def _tail():
        out_ref[0:SPLIT, :] = x_ref[BLK - SPLIT : BLK, :]


def kernel(x, latent_stack):
    return pl.pallas_call(
        _shift_kernel,
        grid=(NBLK,),
        out_shape=jax.ShapeDtypeStruct((STACK, FEAT), jnp.float32),
        in_specs=[
            pl.BlockSpec((BATCH, FEAT), lambda i: (0, 0)),
            pl.BlockSpec((BLK, FEAT), lambda i: (jnp.minimum(i + 1, NBLK - 1), 0)),
        ],
        out_specs=pl.BlockSpec((BLK, FEAT), lambda i: (i, 0)),
        compiler_params=pltpu.CompilerParams(
            dimension_semantics=("parallel",),
        ),
    )(x, latent_stack)


# speedup vs baseline: 27.5260x; 1.2685x over previous
import functools

import jax
import jax.numpy as jnp
from jax import lax
from jax.experimental import pallas as pl
from jax.experimental.pallas import tpu as pltpu
from jax.experimental.pallas import tpu_sc as plsc

BATCH = 1024
STACK = 100000
FEAT = 128
KEEP = STACK - BATCH

NC = 2
NS = 16
NW = NC * NS

KEEP_E = KEEP * FEAT
SHIFT_E = BATCH * FEAT
SPAN_E = KEEP_E // NW
CHUNK_E = 256 * FEAT
NFULL = SPAN_E // CHUNK_E
TAIL_E = SPAN_E - NFULL * CHUNK_E
XB_E = BATCH * FEAT // NW

_mesh = plsc.VectorSubcoreMesh(core_axis_name="c", subcore_axis_name="s")


@functools.partial(
    pl.kernel,
    out_type=jax.ShapeDtypeStruct((STACK * FEAT,), jnp.float32),
    mesh=_mesh,
    scratch_types=[
        pltpu.VMEM((CHUNK_E,), jnp.float32),
        pltpu.VMEM((CHUNK_E,), jnp.float32),
        pltpu.SemaphoreType.DMA,
        pltpu.SemaphoreType.DMA,
        pltpu.SemaphoreType.DMA,
        pltpu.SemaphoreType.DMA,
    ],
)
def _sc_shift(x_hbm, st_hbm, out_hbm, buf0, buf1, rs0, rs1, ws0, ws1):
    wid = lax.axis_index("s") * NC + lax.axis_index("c")
    src_base = SHIFT_E + wid * SPAN_E
    dst_base = wid * SPAN_E
    bufs = (buf0, buf1)
    rsem = (rs0, rs1)
    wsem = (ws0, ws1)

    def rd(i):
        return pltpu.make_async_copy(
            st_hbm.at[pl.ds(src_base + i * CHUNK_E, CHUNK_E)], bufs[i % 2], rsem[i % 2]
        )

    def wr(i):
        return pltpu.make_async_copy(
            bufs[i % 2], out_hbm.at[pl.ds(dst_base + i * CHUNK_E, CHUNK_E)], wsem[i % 2]
        )

    rd(0).start()
    for i in range(NFULL):
        if i + 1 < NFULL:
            if i >= 1:
                wr(i - 1).wait()
            rd(i + 1).start()
        rd(i).wait()
        wr(i).start()
    wr(NFULL - 2).wait()
    wr(NFULL - 1).wait()

    pltpu.sync_copy(
        st_hbm.at[pl.ds(src_base + NFULL * CHUNK_E, TAIL_E)], buf0.at[pl.ds(0, TAIL_E)]
    )
    pltpu.sync_copy(
        buf0.at[pl.ds(0, TAIL_E)], out_hbm.at[pl.ds(dst_base + NFULL * CHUNK_E, TAIL_E)]
    )
    pltpu.sync_copy(x_hbm.at[pl.ds(wid * XB_E, XB_E)], buf1.at[pl.ds(0, XB_E)])
    pltpu.sync_copy(buf1.at[pl.ds(0, XB_E)], out_hbm.at[pl.ds(KEEP_E + wid * XB_E, XB_E)])


def kernel(x, latent_stack):
    flat = _sc_shift(x.reshape(-1), latent_stack.reshape(-1))
    return flat.reshape(STACK, FEAT)


# speedup vs baseline: 28.2665x; 1.0269x over previous
import functools

import jax
import jax.numpy as jnp
from jax import lax
from jax.experimental import pallas as pl
from jax.experimental.pallas import tpu as pltpu
from jax.experimental.pallas import tpu_sc as plsc

BATCH = 1024
STACK = 100000
FEAT = 128
KEEP = STACK - BATCH

NC = 2
NS = 16
NW = NC * NS

KEEP_E = KEEP * FEAT
SHIFT_E = BATCH * FEAT
SPAN_E = KEEP_E // NW
CHUNK_E = 320 * FEAT
NBUF = 3
NFULL = SPAN_E // CHUNK_E
TAIL_E = SPAN_E - NFULL * CHUNK_E
XB_E = BATCH * FEAT // NW

_mesh = plsc.VectorSubcoreMesh(core_axis_name="c", subcore_axis_name="s")


@functools.partial(
    pl.kernel,
    out_type=jax.ShapeDtypeStruct((STACK * FEAT,), jnp.float32),
    mesh=_mesh,
    scratch_types=[
        pltpu.VMEM((CHUNK_E,), jnp.float32),
        pltpu.VMEM((CHUNK_E,), jnp.float32),
        pltpu.VMEM((CHUNK_E,), jnp.float32),
        pltpu.VMEM((XB_E,), jnp.float32),
        pltpu.SemaphoreType.DMA,
        pltpu.SemaphoreType.DMA,
        pltpu.SemaphoreType.DMA,
        pltpu.SemaphoreType.DMA,
        pltpu.SemaphoreType.DMA,
        pltpu.SemaphoreType.DMA,
        pltpu.SemaphoreType.DMA,
    ],
)
def _sc_shift(
    x_hbm, st_hbm, out_hbm, b0, b1, b2, bx, rs0, rs1, rs2, ws0, ws1, ws2, sx
):
    wid = lax.axis_index("s") * NC + lax.axis_index("c")
    src_base = SHIFT_E + wid * SPAN_E
    dst_base = wid * SPAN_E
    bufs = (b0, b1, b2)
    rsem = (rs0, rs1, rs2)
    wsem = (ws0, ws1, ws2)

    def rd(i):
        return pltpu.make_async_copy(
            st_hbm.at[pl.ds(src_base + i * CHUNK_E, CHUNK_E)],
            bufs[i % NBUF],
            rsem[i % NBUF],
        )

    def wr(i):
        return pltpu.make_async_copy(
            bufs[i % NBUF],
            out_hbm.at[pl.ds(dst_base + i * CHUNK_E, CHUNK_E)],
            wsem[i % NBUF],
        )

    x_rd = pltpu.make_async_copy(x_hbm.at[pl.ds(wid * XB_E, XB_E)], bx, sx)
    x_rd.start()

    rd(0).start()
    rd(1).start()
    for i in range(NFULL):
        if i + 2 < NFULL:
            if i >= 1:
                wr(i - 1).wait()
            rd(i + 2).start()
        rd(i).wait()
        wr(i).start()

    wr(NFULL - 3).wait()
    tail_rd = pltpu.make_async_copy(
        st_hbm.at[pl.ds(src_base + NFULL * CHUNK_E, TAIL_E)],
        b0.at[pl.ds(0, TAIL_E)],
        rs0,
    )
    tail_rd.start()
    x_rd.wait()
    x_wr = pltpu.make_async_copy(bx, out_hbm.at[pl.ds(KEEP_E + wid * XB_E, XB_E)], sx)
    x_wr.start()
    tail_rd.wait()
    tail_wr = pltpu.make_async_copy(
        b0.at[pl.ds(0, TAIL_E)],
        out_hbm.at[pl.ds(dst_base + NFULL * CHUNK_E, TAIL_E)],
        rs0,
    )
    tail_wr.start()
    wr(NFULL - 2).wait()
    wr(NFULL - 1).wait()
    x_wr.wait()
    tail_wr.wait()


def kernel(x, latent_stack):
    flat = _sc_shift(x.reshape(-1), latent_stack.reshape(-1))
    return flat.reshape(STACK, FEAT)


# speedup vs baseline: 28.2909x; 1.0009x over previous
import functools

import jax
import jax.numpy as jnp
from jax import lax
from jax.experimental import pallas as pl
from jax.experimental.pallas import tpu as pltpu
from jax.experimental.pallas import tpu_sc as plsc

BATCH = 1024
STACK = 100000
FEAT = 128
KEEP = STACK - BATCH

NC = 2
NS = 16
NW = NC * NS

KEEP_E = KEEP * FEAT
SHIFT_E = BATCH * FEAT
SPAN_E = KEEP_E // NW
CHUNK_E = 320 * FEAT
NBUF = 3
NFULL = SPAN_E // CHUNK_E
TAIL_E = SPAN_E - NFULL * CHUNK_E
XB_E = BATCH * FEAT // NW

_mesh = plsc.VectorSubcoreMesh(core_axis_name="c", subcore_axis_name="s")


@functools.partial(
    pl.kernel,
    out_type=jax.ShapeDtypeStruct((STACK * FEAT,), jnp.float32),
    mesh=_mesh,
    scratch_types=[
        pltpu.VMEM((CHUNK_E,), jnp.float32),
        pltpu.VMEM((CHUNK_E,), jnp.float32),
        pltpu.VMEM((CHUNK_E,), jnp.float32),
        pltpu.VMEM((XB_E,), jnp.float32),
        pltpu.SemaphoreType.DMA,
        pltpu.SemaphoreType.DMA,
        pltpu.SemaphoreType.DMA,
        pltpu.SemaphoreType.DMA,
        pltpu.SemaphoreType.DMA,
        pltpu.SemaphoreType.DMA,
        pltpu.SemaphoreType.DMA,
    ],
    compiler_params=pltpu.CompilerParams(
        disable_bounds_checks=True,
        disable_semaphore_checks=True,
    ),
)
def _sc_shift(
    x_hbm, st_hbm, out_hbm, b0, b1, b2, bx, rs0, rs1, rs2, ws0, ws1, ws2, sx
):
    wid = lax.axis_index("s") * NC + lax.axis_index("c")
    src_base = SHIFT_E + wid * SPAN_E
    dst_base = wid * SPAN_E
    bufs = (b0, b1, b2)
    rsem = (rs0, rs1, rs2)
    wsem = (ws0, ws1, ws2)

    def rd(i):
        return pltpu.make_async_copy(
            st_hbm.at[pl.ds(src_base + i * CHUNK_E, CHUNK_E)],
            bufs[i % NBUF],
            rsem[i % NBUF],
        )

    def wr(i):
        return pltpu.make_async_copy(
            bufs[i % NBUF],
            out_hbm.at[pl.ds(dst_base + i * CHUNK_E, CHUNK_E)],
            wsem[i % NBUF],
        )

    x_rd = pltpu.make_async_copy(x_hbm.at[pl.ds(wid * XB_E, XB_E)], bx, sx)
    x_rd.start()

    rd(0).start()
    rd(1).start()
    for i in range(NFULL):
        if i + 2 < NFULL:
            if i >= 1:
                wr(i - 1).wait()
            rd(i + 2).start()
        rd(i).wait()
        wr(i).start()

    wr(NFULL - 3).wait()
    tail_rd = pltpu.make_async_copy(
        st_hbm.at[pl.ds(src_base + NFULL * CHUNK_E, TAIL_E)],
        b0.at[pl.ds(0, TAIL_E)],
        rs0,
    )
    tail_rd.start()
    x_rd.wait()
    x_wr = pltpu.make_async_copy(bx, out_hbm.at[pl.ds(KEEP_E + wid * XB_E, XB_E)], sx)
    x_wr.start()
    tail_rd.wait()
    tail_wr = pltpu.make_async_copy(
        b0.at[pl.ds(0, TAIL_E)],
        out_hbm.at[pl.ds(dst_base + NFULL * CHUNK_E, TAIL_E)],
        rs0,
    )
    tail_wr.start()
    wr(NFULL - 2).wait()
    wr(NFULL - 1).wait()
    x_wr.wait()
    tail_wr.wait()


def kernel(x, latent_stack):
    flat = _sc_shift(x.reshape(-1), latent_stack.reshape(-1))
    return flat.reshape(STACK, FEAT)
